# baseline (speedup 1.0000x reference)
import functools

import jax
import jax.numpy as jnp
from jax import lax
from jax.experimental import pallas as pl
from jax.experimental.pallas import tpu as pltpu
from jax.experimental.pallas import tpu_sc as plsc

B = 4
S = 8192
H = 1024
LANES = 16
NC = 2
NS = 16
NW = NC * NS
TOK = B * S
TPW = TOK // NW
K = 16
NCHUNK = TPW // K
VECS_PER_ROW = H // LANES


def _sc_kernel(ids_hbm, t1_hbm, t2_hbm, out_hbm,
               idx1, idx2, buf1, buf2, sem1, sem2):
    wid = lax.axis_index("s") * NC + lax.axis_index("c")
    b = wid // (S // TPW)
    s0 = (wid % (S // TPW)) * TPW
    pltpu.sync_copy(ids_hbm.at[pl.ds(b * 2 * S + s0, TPW)], idx1)
    pltpu.sync_copy(ids_hbm.at[pl.ds(b * 2 * S + S + s0, TPW)], idx2)

    def chunk_body(c, carry):
        cp1 = pltpu.async_copy(t1_hbm.at[idx1.at[pl.ds(c * K, K)]], buf1, sem1)
        cp2 = pltpu.async_copy(t2_hbm.at[idx2.at[pl.ds(c * K, K)]], buf2, sem2)
        cp1.wait()
        cp2.wait()

        def add_row(r, carry2):
            def add_vec(j, carry3):
                col = j * LANES
                buf1[r, pl.ds(col, LANES)] = (
                    buf1[r, pl.ds(col, LANES)] + buf2[r, pl.ds(col, LANES)]
                )
                return carry3
            return lax.fori_loop(0, VECS_PER_ROW, add_vec, carry2)

        lax.fori_loop(0, K, add_row, carry)
        pltpu.sync_copy(buf1, out_hbm.at[pl.ds(wid * TPW + c * K, K)])
        return carry

    lax.fori_loop(0, NCHUNK, chunk_body, 0)


@jax.jit
def _run(ids_flat, position_table, block_position_table):
    mesh = plsc.VectorSubcoreMesh(core_axis_name="c", subcore_axis_name="s")
    fn = functools.partial(
        pl.kernel,
        mesh=mesh,
        out_type=jax.ShapeDtypeStruct((TOK, H), jnp.float32),
        scratch_types=[
            pltpu.VMEM((TPW,), jnp.int32),
            pltpu.VMEM((TPW,), jnp.int32),
            pltpu.VMEM((K, H), jnp.float32),
            pltpu.VMEM((K, H), jnp.float32),
            pltpu.SemaphoreType.DMA,
            pltpu.SemaphoreType.DMA,
        ],
    )(_sc_kernel)
    return fn(ids_flat, position_table, block_position_table)


def kernel(position_ids, position_table, block_position_table):
    ids_flat = position_ids.astype(jnp.int32).reshape(-1)
    out = _run(ids_flat, position_table, block_position_table)
    return out.reshape(B, S, H)


# speedup vs baseline: 2.9461x; 2.9461x over previous
import functools

import jax
import jax.numpy as jnp
from jax import lax
from jax.experimental import pallas as pl
from jax.experimental.pallas import tpu as pltpu
from jax.experimental.pallas import tpu_sc as plsc

B = 4
S = 8192
H = 1024
LANES = 16
NC = 2
NS = 16
NW = NC * NS
TOK = B * S
TPW = TOK // NW
K = 16
NCHUNK = TPW // K
NITER = NCHUNK // 2
VECS_PER_ROW = H // LANES


def _add_rows(dst, src_a, src_b):
    def row_body(r, carry):
        for j in range(VECS_PER_ROW):
            col = j * LANES
            dst[r, pl.ds(col, LANES)] = (
                src_a[r, pl.ds(col, LANES)] + src_b[r, pl.ds(col, LANES)]
            )
        return carry
    lax.fori_loop(0, K, row_body, 0)


def _sc_kernel(ids_hbm, t1_hbm, t2_hbm, out_hbm,
               idx1, idx2, a1, a2, b1, b2, o0, o1,
               g0, g1, s0, s1):
    wid = lax.axis_index("s") * NC + lax.axis_index("c")
    b = wid // (S // TPW)
    sbase = (wid % (S // TPW)) * TPW
    pltpu.sync_copy(ids_hbm.at[pl.ds(b * 2 * S + sbase, TPW)], idx1)
    pltpu.sync_copy(ids_hbm.at[pl.ds(b * 2 * S + S + sbase, TPW)], idx2)

    def gathers(c, dst1, dst2, sem):
        pltpu.async_copy(t1_hbm.at[idx1.at[pl.ds(c * K, K)]], dst1, sem)
        pltpu.async_copy(t2_hbm.at[idx2.at[pl.ds(c * K, K)]], dst2, sem)

    def wait_gathers(dst1, dst2, sem):
        pltpu.make_async_copy(t1_hbm.at[pl.ds(0, K)], dst1, sem).wait()
        pltpu.make_async_copy(t2_hbm.at[pl.ds(0, K)], dst2, sem).wait()

    def wait_store(src, sem):
        pltpu.make_async_copy(out_hbm.at[pl.ds(0, K)], src, sem).wait()

    gathers(0, a1, a2, g0)
    gathers(1, b1, b2, g1)

    def body(i, carry):
        c0 = 2 * i
        wait_gathers(a1, a2, g0)

        @pl.when(i > 0)
        def _():
            wait_store(o0, s0)

        _add_rows(o0, a1, a2)

        @pl.when(i < NITER - 1)
        def _():
            gathers(c0 + 2, a1, a2, g0)

        pltpu.async_copy(o0, out_hbm.at[pl.ds(wid * TPW + c0 * K, K)], s0)

        wait_gathers(b1, b2, g1)

        @pl.when(i > 0)
        def _():
            wait_store(o1, s1)

        _add_rows(o1, b1, b2)

        @pl.when(i < NITER - 1)
        def _():
            gathers(c0 + 3, b1, b2, g1)

        pltpu.async_copy(o1, out_hbm.at[pl.ds(wid * TPW + (c0 + 1) * K, K)], s1)
        return carry

    lax.fori_loop(0, NITER, body, 0)
    wait_store(o0, s0)
    wait_store(o1, s1)


@jax.jit
def _run(ids_flat, position_table, block_position_table):
    mesh = plsc.VectorSubcoreMesh(core_axis_name="c", subcore_axis_name="s")
    fn = functools.partial(
        pl.kernel,
        mesh=mesh,
        out_type=jax.ShapeDtypeStruct((TOK, H), jnp.float32),
        scratch_types=[
            pltpu.VMEM((TPW,), jnp.int32),
            pltpu.VMEM((TPW,), jnp.int32),
            pltpu.VMEM((K, H), jnp.float32),
            pltpu.VMEM((K, H), jnp.float32),
            pltpu.VMEM((K, H), jnp.float32),
            pltpu.VMEM((K, H), jnp.float32),
            pltpu.VMEM((K, H), jnp.float32),
            pltpu.VMEM((K, H), jnp.float32),
            pltpu.SemaphoreType.DMA,
            pltpu.SemaphoreType.DMA,
            pltpu.SemaphoreType.DMA,
            pltpu.SemaphoreType.DMA,
        ],
    )(_sc_kernel)
    return fn(ids_flat, position_table, block_position_table)


def kernel(position_ids, position_table, block_position_table):
    ids_flat = position_ids.astype(jnp.int32).reshape(-1)
    out = _run(ids_flat, position_table, block_position_table)
    return out.reshape(B, S, H)


# speedup vs baseline: 2.9990x; 1.0180x over previous
import functools

import jax
import jax.numpy as jnp
from jax import lax
from jax.experimental import pallas as pl
from jax.experimental.pallas import tpu as pltpu
from jax.experimental.pallas import tpu_sc as plsc

B = 4
S = 8192
H = 1024
LANES = 16
NC = 2
NS = 16
NW = NC * NS
TOK = B * S
TPW = TOK // NW
K = 8
NCHUNK = TPW // K
NBUF = 4
NITER = NCHUNK // NBUF
VECS_PER_ROW = H // LANES


def _add_rows_inplace(dst, src):
    def row_body(r, carry):
        for j in range(VECS_PER_ROW):
            col = j * LANES
            dst[r, pl.ds(col, LANES)] = (
                dst[r, pl.ds(col, LANES)] + src[r, pl.ds(col, LANES)]
            )
        return carry
    lax.fori_loop(0, K, row_body, 0)


def _sc_kernel(ids_hbm, t1_hbm, t2_hbm, out_hbm,
               idx1, idx2, *bufs_and_sems):
    a = bufs_and_sems[0:NBUF]
    bsuf = bufs_and_sems[NBUF:2 * NBUF]
    gsem = bufs_and_sems[2 * NBUF:3 * NBUF]
    ssem = bufs_and_sems[3 * NBUF:4 * NBUF]

    wid = lax.axis_index("s") * NC + lax.axis_index("c")
    b = wid // (S // TPW)
    sbase = (wid % (S // TPW)) * TPW
    pltpu.sync_copy(ids_hbm.at[pl.ds(b * 2 * S + sbase, TPW)], idx1)
    pltpu.sync_copy(ids_hbm.at[pl.ds(b * 2 * S + S + sbase, TPW)], idx2)

    def gathers(c, t):
        pltpu.async_copy(t1_hbm.at[idx1.at[pl.ds(c * K, K)]], a[t], gsem[t])
        pltpu.async_copy(t2_hbm.at[idx2.at[pl.ds(c * K, K)]], bsuf[t], gsem[t])

    def wait_gathers(t):
        pltpu.make_async_copy(t1_hbm.at[pl.ds(0, K)], a[t], gsem[t]).wait()
        pltpu.make_async_copy(t2_hbm.at[pl.ds(0, K)], bsuf[t], gsem[t]).wait()

    def wait_store(t):
        pltpu.make_async_copy(out_hbm.at[pl.ds(0, K)], a[t], ssem[t]).wait()

    gathers(0, 0)
    gathers(1, 1)

    def body(i, carry):
        for t in range(NBUF):
            c = NBUF * i + t
            wait_gathers(t)

            nt = (t + 2) % NBUF

            @pl.when(c >= 2)
            def _():
                wait_store(nt)

            @pl.when(c < NCHUNK - 2)
            def _():
                gathers(c + 2, nt)

            _add_rows_inplace(a[t], bsuf[t])
            pltpu.async_copy(a[t], out_hbm.at[pl.ds(wid * TPW + c * K, K)],
                             ssem[t])
        return carry

    lax.fori_loop(0, NITER, body, 0)
    wait_store((NCHUNK - 2) % NBUF)
    wait_store((NCHUNK - 1) % NBUF)


@jax.jit
def _run(ids_flat, position_table, block_position_table):
    mesh = plsc.VectorSubcoreMesh(core_axis_name="c", subcore_axis_name="s")
    fn = functools.partial(
        pl.kernel,
        mesh=mesh,
        out_type=jax.ShapeDtypeStruct((TOK, H), jnp.float32),
        scratch_types=(
            [pltpu.VMEM((TPW,), jnp.int32)] * 2
            + [pltpu.VMEM((K, H), jnp.float32)] * (2 * NBUF)
            + [pltpu.SemaphoreType.DMA] * (2 * NBUF)
        ),
    )(_sc_kernel)
    return fn(ids_flat, position_table, block_position_table)


def kernel(position_ids, position_table, block_position_table):
    ids_flat = position_ids.astype(jnp.int32).reshape(-1)
    out = _run(ids_flat, position_table, block_position_table)
    return out.reshape(B, S, H)


# speedup vs baseline: 3.0517x; 1.0176x over previous
import functools

import jax
import jax.numpy as jnp
from jax import lax
from jax.experimental import pallas as pl
from jax.experimental.pallas import tpu as pltpu
from jax.experimental.pallas import tpu_sc as plsc

B = 4
S = 8192
H = 1024
LANES = 16
NC = 2
NS = 16
NW = NC * NS
TOK = B * S
TPW = TOK // NW
K = 8
NCHUNK = TPW // K
NBUF = 4
NITER = NCHUNK // NBUF
VECS_PER_ROW = H // LANES


def _add_rows_inplace(dst, src):
    def row_body(r, carry):
        for j in range(VECS_PER_ROW):
            col = j * LANES
            dst[r, pl.ds(col, LANES)] = (
                dst[r, pl.ds(col, LANES)] + src[r, pl.ds(col, LANES)]
            )
        return carry
    lax.fori_loop(0, K, row_body, 0)


def _sc_kernel(ids_hbm, t1_hbm, t2_hbm, out_hbm,
               idx1, idx2, *bufs_and_sems):
    a = bufs_and_sems[0:NBUF]
    bsuf = bufs_and_sems[NBUF:2 * NBUF]
    gsem = bufs_and_sems[2 * NBUF:3 * NBUF]
    ssem = bufs_and_sems[3 * NBUF:4 * NBUF]

    wid = lax.axis_index("s") * NC + lax.axis_index("c")
    b = wid // (S // TPW)
    sbase = (wid % (S // TPW)) * TPW
    pltpu.sync_copy(ids_hbm.at[pl.ds(b * 2 * S + sbase, TPW)], idx1)
    pltpu.sync_copy(ids_hbm.at[pl.ds(b * 2 * S + S + sbase, TPW)], idx2)

    def gathers(c, t):
        pltpu.async_copy(t1_hbm.at[idx1.at[pl.ds(c * K, K)]], a[t], gsem[t])
        pltpu.async_copy(t2_hbm.at[idx2.at[pl.ds(c * K, K)]], bsuf[t], gsem[t])

    def wait_gathers(t):
        pltpu.make_async_copy(t1_hbm.at[pl.ds(0, K)], a[t], gsem[t]).wait()
        pltpu.make_async_copy(t2_hbm.at[pl.ds(0, K)], bsuf[t], gsem[t]).wait()

    def wait_store(t):
        pltpu.make_async_copy(out_hbm.at[pl.ds(0, K)], a[t], ssem[t]).wait()

    gathers(0, 0)
    gathers(1, 1)
    gathers(2, 2)

    def body(i, carry):
        for t in range(NBUF):
            c = NBUF * i + t
            wait_gathers(t)

            nt = (t + 3) % NBUF

            @pl.when(c >= 1)
            def _():
                wait_store(nt)

            @pl.when(c < NCHUNK - 3)
            def _():
                gathers(c + 3, nt)

            _add_rows_inplace(a[t], bsuf[t])
            pltpu.async_copy(a[t], out_hbm.at[pl.ds(wid * TPW + c * K, K)],
                             ssem[t])
        return carry

    lax.fori_loop(0, NITER, body, 0)
    wait_store((NCHUNK - 1) % NBUF)


@jax.jit
def _run(ids_flat, position_table, block_position_table):
    mesh = plsc.VectorSubcoreMesh(core_axis_name="c", subcore_axis_name="s")
    fn = functools.partial(
        pl.kernel,
        mesh=mesh,
        out_type=jax.ShapeDtypeStruct((TOK, H), jnp.float32),
        scratch_types=(
            [pltpu.VMEM((TPW,), jnp.int32)] * 2
            + [pltpu.VMEM((K, H), jnp.float32)] * (2 * NBUF)
            + [pltpu.SemaphoreType.DMA] * (2 * NBUF)
        ),
    )(_sc_kernel)
    return fn(ids_flat, position_table, block_position_table)


def kernel(position_ids, position_table, block_position_table):
    ids_flat = position_ids.astype(jnp.int32).reshape(-1)
    out = _run(ids_flat, position_table, block_position_table)
    return out.reshape(B, S, H)
